# baseline (speedup 1.0000x reference)
import functools

import jax
import jax.numpy as jnp
from jax import lax
from jax.experimental import pallas as pl
from jax.experimental.pallas import tpu as pltpu
from jax.experimental.pallas import tpu_sc as plsc

NC = 2
NS = 16
NW = NC * NS
L = 16
B = 16384
D = 64
BPW = B // NW
GSZ = 128
NG = BPW // GSZ


_GDN = lax.GatherDimensionNumbers(
    offset_dims=(), collapsed_slice_dims=(0,), start_index_map=(0,))


def _lane_perm(x, perm):
    return lax.gather(
        x, perm[:, None], dimension_numbers=_GDN, slice_sizes=(1,),
        mode=lax.GatherScatterMode.PROMISE_IN_BOUNDS)


def _body(user_h, item_h, ue_h, ie_h, ub_h, ib_h, out_h,
          idx_u, idx_i, rows_u, rows_i, bu, bi, outv, sem):
    wid = lax.axis_index("s") * NC + lax.axis_index("c")

    pltpu.sync_copy(user_h.at[wid], idx_u)
    pltpu.sync_copy(item_h.at[wid], idx_i)

    copies = []
    for j in range(NG):
        dst = pl.ds(j * GSZ, GSZ)
        copies.append(pltpu.async_copy(ue_h.at[idx_u.at[j]], rows_u.at[dst], sem))
        copies.append(pltpu.async_copy(ie_h.at[idx_i.at[j]], rows_i.at[dst], sem))
        copies.append(pltpu.async_copy(ub_h.at[idx_u.at[j]], bu.at[dst], sem))
        copies.append(pltpu.async_copy(ib_h.at[idx_i.at[j]], bi.at[dst], sem))
    for c in copies:
        c.wait()

    lane = lax.iota(jnp.int32, 16)
    perms = [lane ^ (1 << p) for p in range(4)]

    def chunk(c, carry):
        row0 = c * L
        res = jnp.zeros((L,), jnp.float32)
        for r in range(L):
            row = row0 + r
            acc = rows_u[row, pl.ds(0, L)] * rows_i[row, pl.ds(0, L)]
            for k in range(1, D // L):
                acc = acc + rows_u[row, pl.ds(k * L, L)] * rows_i[row, pl.ds(k * L, L)]
            for p in perms:
                acc = acc + _lane_perm(acc, p)
            res = jnp.where(lane == r, acc, res)
        res = res + bu[pl.ds(row0, L)] + bi[pl.ds(row0, L)]
        outv[pl.ds(row0, L)] = res
        return carry

    lax.fori_loop(0, BPW // L, chunk, 0)

    pltpu.sync_copy(outv, out_h.at[wid])


@jax.jit
def _run(user, item, ue, ie, ub, ib):
    k = pl.kernel(
        _body,
        mesh=plsc.VectorSubcoreMesh(core_axis_name="c", subcore_axis_name="s"),
        compiler_params=pltpu.CompilerParams(use_tc_tiling_on_sc=False),
        out_type=jax.ShapeDtypeStruct((NW, BPW), jnp.float32),
        scratch_types=[
            pltpu.VMEM((NG, GSZ), jnp.int32),
            pltpu.VMEM((NG, GSZ), jnp.int32),
            pltpu.VMEM((BPW, D), jnp.float32),
            pltpu.VMEM((BPW, D), jnp.float32),
            pltpu.VMEM((BPW,), jnp.float32),
            pltpu.VMEM((BPW,), jnp.float32),
            pltpu.VMEM((BPW,), jnp.float32),
            pltpu.SemaphoreType.DMA,
        ],
    )
    return k(user, item, ue, ie, ub, ib)


def kernel(user, item, user_embedding, item_embedding, user_bias, item_bias):
    u = user.astype(jnp.int32).reshape(NW, NG, GSZ)
    it = item.astype(jnp.int32).reshape(NW, NG, GSZ)
    ub = user_bias.reshape(-1)
    ib = item_bias.reshape(-1)
    out = _run(u, it, user_embedding, item_embedding, ub, ib)
    return out.reshape(B)
